# baseline (speedup 1.0000x reference)
import functools

import jax
import jax.numpy as jnp
from jax import lax
from jax.experimental import pallas as pl
from jax.experimental.pallas import tpu as pltpu
from jax.experimental.pallas import tpu_sc as plsc

_NC, _NS = 2, 16
_NW = _NC * _NS
_CHUNK = 80


def _mm_bias_body(x_ref, w_ref, b_ref, o_ref):
    o_ref[...] = (
        jnp.dot(x_ref[...], w_ref[...], preferred_element_type=jnp.float32)
        + b_ref[...]
    )


def _tc_mm_bias(x, W, b, block_rows=2000):
    n, fin = x.shape
    fout = W.shape[1]
    grid = n // block_rows
    return pl.pallas_call(
        _mm_bias_body,
        grid=(grid,),
        in_specs=[
            pl.BlockSpec((block_rows, fin), lambda i: (i, 0)),
            pl.BlockSpec((fin, fout), lambda i: (0, 0)),
            pl.BlockSpec((fout,), lambda i: (0,)),
        ],
        out_specs=pl.BlockSpec((block_rows, fout), lambda i: (i, 0)),
        out_shape=jax.ShapeDtypeStruct((n, fout), jnp.float32),
    )(x, W, b)


def _kmlp_body(h0_ref, h1_ref, b0_ref, w1_ref, b1_ref, w2_ref, b2_ref, o_ref):
    h = jnp.tanh(h0_ref[:, :64] + h1_ref[:, :64] + b0_ref[...])
    h = jax.nn.relu(
        jnp.dot(h, w1_ref[...], preferred_element_type=jnp.float32) + b1_ref[...]
    )
    k = jnp.dot(h, w2_ref[...], preferred_element_type=jnp.float32) + b2_ref[...]
    o_ref[...] = jnp.concatenate([k, jnp.zeros_like(k)], axis=1)


def _tc_node_mlp(h0, h1, b0, W1, b1, W2, b2, block_rows=2000):
    n = h0.shape[0]
    f = 64
    grid = n // block_rows
    return pl.pallas_call(
        _kmlp_body,
        grid=(grid,),
        in_specs=[
            pl.BlockSpec((block_rows, 2 * f), lambda i: (i, 0)),
            pl.BlockSpec((block_rows, 2 * f), lambda i: (i, 0)),
            pl.BlockSpec((f,), lambda i: (0,)),
            pl.BlockSpec((f, f), lambda i: (0, 0)),
            pl.BlockSpec((f,), lambda i: (0,)),
            pl.BlockSpec((f, f), lambda i: (0, 0)),
            pl.BlockSpec((f,), lambda i: (0,)),
        ],
        out_specs=pl.BlockSpec((block_rows, 2 * f), lambda i: (i, 0)),
        out_shape=jax.ShapeDtypeStruct((n, 2 * f), jnp.float32),
    )(h0, h1, b0, W1, b1, W2, b2)


def _edge_mlp_body(g_ref, w1_ref, b1_ref, w2_ref, b2_ref, o_ref):
    t = jnp.tanh(g_ref[:, :64])
    h = jax.nn.relu(
        jnp.dot(t, w1_ref[...], preferred_element_type=jnp.float32) + b1_ref[...]
    )
    o_ref[...] = (
        jnp.dot(h, w2_ref[...], preferred_element_type=jnp.float32) + b2_ref[...]
    )


def _tc_edge_mlp(t0x, W1, b1, W2, b2, block_rows=8000):
    e = t0x.shape[0]
    f = 64
    grid = e // block_rows
    return pl.pallas_call(
        _edge_mlp_body,
        grid=(grid,),
        in_specs=[
            pl.BlockSpec((block_rows, 2 * f), lambda i: (i, 0)),
            pl.BlockSpec((f, f), lambda i: (0, 0)),
            pl.BlockSpec((f,), lambda i: (0,)),
            pl.BlockSpec((f, f), lambda i: (0, 0)),
            pl.BlockSpec((f,), lambda i: (0,)),
        ],
        out_specs=pl.BlockSpec((block_rows, f), lambda i: (i, 0)),
        out_shape=jax.ShapeDtypeStruct((e, f), jnp.float32),
    )(t0x, W1, b1, W2, b2)


def _final_dense_body(e0_ref, e1_ref, hw_ref, hb_ref, dw_ref, db_ref,
                      dd_ref, pre_ref):
    en = e0_ref[:, :64] + e1_ref[:, :64]
    dh = jnp.dot(en, hw_ref[...], preferred_element_type=jnp.float32) + hb_ref[...]
    dd_ref[...] = (
        jnp.dot(dh, dw_ref[...], preferred_element_type=jnp.float32) + db_ref[...]
    )
    half = dh.shape[1] // 2
    pre_ref[...] = jnp.concatenate([dh[:, half:], -dh[:, :half]], axis=1)


def _tc_final_dense(e0, e1, H_W, H_b, D_W, D_b, block_rows=2000):
    n = e0.shape[0]
    f = 64
    fo = H_W.shape[1]
    grid = n // block_rows
    return pl.pallas_call(
        _final_dense_body,
        grid=(grid,),
        in_specs=[
            pl.BlockSpec((block_rows, 2 * f), lambda i: (i, 0)),
            pl.BlockSpec((block_rows, 2 * f), lambda i: (i, 0)),
            pl.BlockSpec((f, fo), lambda i: (0, 0)),
            pl.BlockSpec((fo,), lambda i: (0,)),
            pl.BlockSpec((fo, fo), lambda i: (0, 0)),
            pl.BlockSpec((fo,), lambda i: (0,)),
        ],
        out_specs=[
            pl.BlockSpec((block_rows, fo), lambda i: (i, 0)),
            pl.BlockSpec((block_rows, fo), lambda i: (i, 0)),
        ],
        out_shape=[
            jax.ShapeDtypeStruct((n, fo), jnp.float32),
            jax.ShapeDtypeStruct((n, fo), jnp.float32),
        ],
    )(e0, e1, H_W, H_b, D_W, D_b)


def _combine_body(pre_ref, d0_ref, d1_ref, o_ref):
    o_ref[...] = pre_ref[...] - d0_ref[...] - d1_ref[...]


def _tc_combine(pre, d0, d1, block_rows=2000):
    n, f = pre.shape
    grid = n // block_rows
    return pl.pallas_call(
        _combine_body,
        grid=(grid,),
        in_specs=[pl.BlockSpec((block_rows, f), lambda i: (i, 0))] * 3,
        out_specs=pl.BlockSpec((block_rows, f), lambda i: (i, 0)),
        out_shape=jax.ShapeDtypeStruct((n, f), jnp.float32),
    )(pre, d0, d1)


def _sc_mesh():
    return plsc.VectorSubcoreMesh(
        core_axis_name="c", subcore_axis_name="s",
        num_cores=_NC, num_subcores=_NS)


def _seg_sum(table, src, dst, n):
    d = table.shape[1]
    e = src.shape[0]
    per_w = e // _NW
    nchunks = per_w // _CHUNK
    nzt = 10
    rows_per_tile = n // nzt
    zeros = jnp.zeros((rows_per_tile, d), jnp.float32)

    @functools.partial(
        pl.kernel,
        out_type=jax.ShapeDtypeStruct((_NC, n, d), jnp.float32),
        mesh=_sc_mesh(),
        scratch_types=[
            pltpu.VMEM((4, _CHUNK), jnp.int32),
            pltpu.VMEM((4, _CHUNK), jnp.int32),
            pltpu.VMEM((2, _CHUNK, d), jnp.float32),
            pltpu.VMEM_SHARED((n, d), jnp.float32),
        ] + [pltpu.SemaphoreType.DMA] * 12,
    )
    def seg_kernel(table_hbm, src_hbm, dst_hbm, zeros_hbm, out_hbm,
                   idxg_v, idxs_v, rows_v, acc_sh, *sems):
        c = lax.axis_index("c")
        s = lax.axis_index("s")
        wid = s * _NC + c
        base = wid * per_w
        gsem = sems[0:2]
        ssem = sems[2:4]
        igsem = sems[4:8]
        issem = sems[8:12]

        def ldg(ci, sl):
            return pltpu.make_async_copy(
                src_hbm.at[pl.ds(base + ci * _CHUNK, _CHUNK)],
                idxg_v.at[sl], igsem[sl])

        def lds(ci, sl):
            return pltpu.make_async_copy(
                dst_hbm.at[pl.ds(base + ci * _CHUNK, _CHUNK)],
                idxs_v.at[sl], issem[sl])

        def gather(sl, b):
            return pltpu.make_async_copy(
                table_hbm.at[idxg_v.at[sl]], rows_v.at[b], gsem[b])

        def scatter(sl, b):
            return pltpu.make_async_copy(
                rows_v.at[b], acc_sh.at[idxs_v.at[sl]], ssem[b])

        @pl.when(s < nzt)
        def _():
            pltpu.sync_copy(zeros_hbm,
                            acc_sh.at[pl.ds(s * rows_per_tile, rows_per_tile)])
        for u in range(4):
            ldg(u, u).start()
            lds(u, u).start()
        for b in range(2):
            ldg(b, b).wait()
            lds(b, b).wait()
            gather(b, b).start()
        plsc.subcore_barrier()

        @pl.loop(0, (nchunks + 3) // 4)
        def _(cq):
            for u in range(4):
                ci = cq * 4 + u
                b = u % 2
                ob = 1 - b
                osl = (u + 1) % 4
                psl = (u + 3) % 4

                @pl.when(ci < nchunks)
                def _():
                    gather(u, b).wait()
                    scatter(u, b).start(add=True)

                @pl.when(jnp.logical_and(ci >= 1, ci + 1 < nchunks))
                def _():
                    scatter(psl, ob).wait()
                    ldg(ci + 1, osl).wait()
                    lds(ci + 1, osl).wait()
                    gather(osl, ob).start()

                    @pl.when(ci + 3 < nchunks)
                    def _():
                        ldg(ci + 3, psl).start()
                        lds(ci + 3, psl).start()

        scatter((nchunks - 2) % 4, (nchunks - 2) % 2).wait()
        scatter((nchunks - 1) % 4, (nchunks - 1) % 2).wait()
        plsc.subcore_barrier()

        @pl.when(s < nzt)
        def _():
            pltpu.sync_copy(acc_sh.at[pl.ds(s * rows_per_tile, rows_per_tile)],
                            out_hbm.at[c, pl.ds(s * rows_per_tile, rows_per_tile)])

    out = seg_kernel(table, src, dst, zeros)
    return out[0], out[1]


def _seg_sum_and_edge_gather(ts, t2, src, dst, n):
    d = ts.shape[1]
    e = src.shape[0]
    per_w = e // _NW
    nchunks = per_w // _CHUNK
    nzt = 10
    rows_per_tile = n // nzt
    zeros = jnp.zeros((rows_per_tile, d), jnp.float32)

    @functools.partial(
        pl.kernel,
        out_type=(jax.ShapeDtypeStruct((_NC, n, d), jnp.float32),
                  jax.ShapeDtypeStruct((e, d), jnp.float32)),
        mesh=_sc_mesh(),
        scratch_types=[
            pltpu.VMEM((4, _CHUNK), jnp.int32),
            pltpu.VMEM((4, _CHUNK), jnp.int32),
            pltpu.VMEM((2, _CHUNK, d), jnp.float32),
            pltpu.VMEM((2, _CHUNK, d), jnp.float32),
            pltpu.VMEM_SHARED((n, d), jnp.float32),
        ] + [pltpu.SemaphoreType.DMA] * 16,
    )
    def fused_kernel(ts_hbm, t2_hbm, src_hbm, dst_hbm, zeros_hbm,
                     out_hbm, t0x_hbm,
                     idxg_v, idxs_v, r1_v, r2_v, acc_sh, *sems):
        c = lax.axis_index("c")
        s = lax.axis_index("s")
        wid = s * _NC + c
        base = wid * per_w
        g1sem = sems[0:2]
        g2sem = sems[2:4]
        ssem = sems[4:6]
        wsem = sems[6:8]
        igsem = sems[8:12]
        issem = sems[12:16]

        def ldg(ci, sl):
            return pltpu.make_async_copy(
                src_hbm.at[pl.ds(base + ci * _CHUNK, _CHUNK)],
                idxg_v.at[sl], igsem[sl])

        def lds(ci, sl):
            return pltpu.make_async_copy(
                dst_hbm.at[pl.ds(base + ci * _CHUNK, _CHUNK)],
                idxs_v.at[sl], issem[sl])

        def gather1(sl, b):
            return pltpu.make_async_copy(
                ts_hbm.at[idxg_v.at[sl]], r1_v.at[b], g1sem[b])

        def gather2(sl, b):
            return pltpu.make_async_copy(
                t2_hbm.at[idxs_v.at[sl]], r2_v.at[b], g2sem[b])

        def scatter(sl, b):
            return pltpu.make_async_copy(
                r1_v.at[b], acc_sh.at[idxs_v.at[sl]], ssem[b])

        def twrite(ci, b):
            return pltpu.make_async_copy(
                r2_v.at[b], t0x_hbm.at[pl.ds(base + ci * _CHUNK, _CHUNK)],
                wsem[b])

        @pl.when(s < nzt)
        def _():
            pltpu.sync_copy(zeros_hbm,
                            acc_sh.at[pl.ds(s * rows_per_tile, rows_per_tile)])
        for u in range(4):
            ldg(u, u).start()
            lds(u, u).start()
        for b in range(2):
            ldg(b, b).wait()
            lds(b, b).wait()
            gather1(b, b).start()
            gather2(b, b).start()
        plsc.subcore_barrier()

        @pl.loop(0, (nchunks + 3) // 4)
        def _(cq):
            for u in range(4):
                ci = cq * 4 + u
                b = u % 2
                ob = 1 - b
                osl = (u + 1) % 4
                psl = (u + 3) % 4

                @pl.when(ci < nchunks)
                def _():
                    gather1(u, b).wait()
                    scatter(u, b).start(add=True)
                    gather2(u, b).wait()

                    @pl.loop(0, _CHUNK, unroll=8)
                    def _(ri):
                        for j in range(d // 2 // 16):
                            lo = pl.ds(j * 16, 16)
                            hi = pl.ds(d // 2 + j * 16, 16)
                            r2_v[b, ri, lo] = r1_v[b, ri, hi] + r2_v[b, ri, lo]

                    twrite(ci, b).start()

                @pl.when(jnp.logical_and(ci >= 1, ci + 1 < nchunks))
                def _():
                    scatter(psl, ob).wait()
                    twrite(ci - 1, ob).wait()
                    ldg(ci + 1, osl).wait()
                    lds(ci + 1, osl).wait()
                    gather1(osl, ob).start()
                    gather2(osl, ob).start()

                    @pl.when(ci + 3 < nchunks)
                    def _():
                        ldg(ci + 3, psl).start()
                        lds(ci + 3, psl).start()

        for ci in (nchunks - 2, nchunks - 1):
            b = ci % 2
            scatter(ci % 4, b).wait()
            twrite(ci, b).wait()
        plsc.subcore_barrier()

        @pl.when(s < nzt)
        def _():
            pltpu.sync_copy(acc_sh.at[pl.ds(s * rows_per_tile, rows_per_tile)],
                            out_hbm.at[c, pl.ds(s * rows_per_tile, rows_per_tile)])

    return fused_kernel(ts, t2, src, dst, zeros)


def _gather_mul_seg_sum(kpad, p, src, dst, n):
    d = kpad.shape[1]
    dp = p.shape[1]
    e = src.shape[0]
    per_w = e // _NW
    nchunks = per_w // _CHUNK
    nzt = 10
    rows_per_tile = n // nzt
    zeros = jnp.zeros((rows_per_tile, d), jnp.float32)

    @functools.partial(
        pl.kernel,
        out_type=jax.ShapeDtypeStruct((_NC, n, d), jnp.float32),
        mesh=_sc_mesh(),
        scratch_types=[
            pltpu.VMEM((4, _CHUNK), jnp.int32),
            pltpu.VMEM((4, _CHUNK), jnp.int32),
            pltpu.VMEM((2, _CHUNK, d), jnp.float32),
            pltpu.VMEM((2, _CHUNK, dp), jnp.float32),
            pltpu.VMEM_SHARED((n, d), jnp.float32),
        ] + [pltpu.SemaphoreType.DMA] * 14,
    )
    def gm_kernel(k_hbm, p_hbm, src_hbm, dst_hbm, zeros_hbm, out_hbm,
                  idxg_v, idxs_v, rows_v, pv_v, acc_sh, *sems):
        c = lax.axis_index("c")
        s = lax.axis_index("s")
        wid = s * _NC + c
        base = wid * per_w
        ksem = sems[0:2]
        psem = sems[2:4]
        ssem = sems[4:6]
        igsem = sems[6:10]
        issem = sems[10:14]

        def ldg(ci, sl):
            return pltpu.make_async_copy(
                src_hbm.at[pl.ds(base + ci * _CHUNK, _CHUNK)],
                idxg_v.at[sl], igsem[sl])

        def lds(ci, sl):
            return pltpu.make_async_copy(
                dst_hbm.at[pl.ds(base + ci * _CHUNK, _CHUNK)],
                idxs_v.at[sl], issem[sl])

        def gather(sl, b):
            return pltpu.make_async_copy(
                k_hbm.at[idxg_v.at[sl]], rows_v.at[b], ksem[b])

        def pread(ci, b):
            return pltpu.make_async_copy(
                p_hbm.at[pl.ds(base + ci * _CHUNK, _CHUNK)], pv_v.at[b],
                psem[b])

        def scatter(sl, b):
            return pltpu.make_async_copy(
                rows_v.at[b], acc_sh.at[idxs_v.at[sl]], ssem[b])

        @pl.when(s < nzt)
        def _():
            pltpu.sync_copy(zeros_hbm,
                            acc_sh.at[pl.ds(s * rows_per_tile, rows_per_tile)])
        for u in range(4):
            ldg(u, u).start()
            lds(u, u).start()
        for b in range(2):
            ldg(b, b).wait()
            lds(b, b).wait()
            gather(b, b).start()
            pread(b, b).start()
        plsc.subcore_barrier()

        @pl.loop(0, (nchunks + 3) // 4)
        def _(cq):
            for u in range(4):
                ci = cq * 4 + u
                b = u % 2
                ob = 1 - b
                osl = (u + 1) % 4
                psl = (u + 3) % 4

                @pl.when(ci < nchunks)
                def _():
                    gather(u, b).wait()
                    pread(ci, b).wait()

                    @pl.loop(0, _CHUNK, unroll=8)
                    def _(ri):
                        for j in range(dp // 16):
                            sl = pl.ds(j * 16, 16)
                            rows_v[b, ri, sl] = (
                                rows_v[b, ri, sl] * pv_v[b, ri, sl])

                    scatter(u, b).start(add=True)

                @pl.when(jnp.logical_and(ci >= 1, ci + 1 < nchunks))
                def _():
                    scatter(psl, ob).wait()
                    ldg(ci + 1, osl).wait()
                    lds(ci + 1, osl).wait()
                    gather(osl, ob).start()
                    pread(ci + 1, ob).start()

                    @pl.when(ci + 3 < nchunks)
                    def _():
                        ldg(ci + 3, psl).start()
                        lds(ci + 3, psl).start()

        scatter((nchunks - 2) % 4, (nchunks - 2) % 2).wait()
        scatter((nchunks - 1) % 4, (nchunks - 1) % 2).wait()
        plsc.subcore_barrier()

        @pl.when(s < nzt)
        def _():
            pltpu.sync_copy(acc_sh.at[pl.ds(s * rows_per_tile, rows_per_tile)],
                            out_hbm.at[c, pl.ds(s * rows_per_tile, rows_per_tile)])

    out = gm_kernel(kpad, p, src, dst, zeros)
    return out[0], out[1]


def kernel(x, src, dst, encK_W, encK_b, encP1_W, encP1_b, encP2_W, encP2_b,
           Kf0_W, Kf0_b, Kf1_W, Kf1_b, Kf2_W, Kf2_b,
           Uf0_W, Uf0_b, Uf1_W, Uf1_b, Uf2_W, Uf2_b, H_W, H_b, D_W, D_b):
    n, in_dim = x.shape

    WB = encK_W @ Kf0_W
    bB = encK_b @ Kf0_W
    W1 = encP1_W @ Uf0_W
    b1 = encP1_b @ Uf0_W + Uf0_b
    W2 = encP2_W @ Uf0_W
    b2 = encP2_b @ Uf0_W
    Wcat = jnp.concatenate(
        [WB, W1, W2, jnp.zeros_like(W2)], axis=1)
    bcat = jnp.concatenate([bB, b1, b2, jnp.zeros_like(b2)], axis=0)

    pre_all = _tc_mm_bias(x, Wcat, bcat)
    ts = lax.slice(pre_all, (0, 0), (n, 128))
    t2 = lax.slice(pre_all, (0, 128), (n, 256))

    part, t0x = _seg_sum_and_edge_gather(ts, t2, src, dst, n)
    kpad = _tc_node_mlp(part[0], part[1],
                        Kf0_b, Kf1_W, Kf1_b, Kf2_W, Kf2_b)

    P = _tc_edge_mlp(t0x, Uf1_W, Uf1_b, Uf2_W, Uf2_b)

    e0, e1 = _gather_mul_seg_sum(kpad, P, src, dst, n)
    dd, pre = _tc_final_dense(e0, e1, H_W, H_b, D_W, D_b)

    d0, d1 = _seg_sum(dd, src, dst, n)
    return _tc_combine(pre, d0, d1)


# speedup vs baseline: 1.2746x; 1.2746x over previous
import functools

import jax
import jax.numpy as jnp
from jax import lax
from jax.experimental import pallas as pl
from jax.experimental.pallas import tpu as pltpu
from jax.experimental.pallas import tpu_sc as plsc

_NC, _NS = 2, 16
_NW = _NC * _NS
_CHUNK = 80


def _mm_bias_body(x_ref, w_ref, b_ref, o_ref):
    o_ref[...] = (
        jnp.dot(x_ref[...], w_ref[...], preferred_element_type=jnp.float32)
        + b_ref[...]
    )


def _tc_mm_bias(x, W, b, block_rows=2000):
    n, fin = x.shape
    fout = W.shape[1]
    grid = n // block_rows
    return pl.pallas_call(
        _mm_bias_body,
        grid=(grid,),
        in_specs=[
            pl.BlockSpec((block_rows, fin), lambda i: (i, 0)),
            pl.BlockSpec((fin, fout), lambda i: (0, 0)),
            pl.BlockSpec((fout,), lambda i: (0,)),
        ],
        out_specs=pl.BlockSpec((block_rows, fout), lambda i: (i, 0)),
        out_shape=jax.ShapeDtypeStruct((n, fout), jnp.float32),
    )(x, W, b)


def _kmlp_body(h0_ref, h1_ref, b0_ref, w1_ref, b1_ref, w2_ref, b2_ref, o_ref):
    h = jnp.tanh(h0_ref[:, :64] + h1_ref[:, :64] + b0_ref[...])
    h = jax.nn.relu(
        jnp.dot(h, w1_ref[...], preferred_element_type=jnp.float32) + b1_ref[...]
    )
    k = jnp.dot(h, w2_ref[...], preferred_element_type=jnp.float32) + b2_ref[...]
    o_ref[...] = jnp.concatenate([k, jnp.zeros_like(k)], axis=1)


def _tc_node_mlp(h0, h1, b0, W1, b1, W2, b2, block_rows=2000):
    n = h0.shape[0]
    f = 64
    grid = n // block_rows
    return pl.pallas_call(
        _kmlp_body,
        grid=(grid,),
        in_specs=[
            pl.BlockSpec((block_rows, 2 * f), lambda i: (i, 0)),
            pl.BlockSpec((block_rows, 2 * f), lambda i: (i, 0)),
            pl.BlockSpec((f,), lambda i: (0,)),
            pl.BlockSpec((f, f), lambda i: (0, 0)),
            pl.BlockSpec((f,), lambda i: (0,)),
            pl.BlockSpec((f, f), lambda i: (0, 0)),
            pl.BlockSpec((f,), lambda i: (0,)),
        ],
        out_specs=pl.BlockSpec((block_rows, 2 * f), lambda i: (i, 0)),
        out_shape=jax.ShapeDtypeStruct((n, 2 * f), jnp.float32),
    )(h0, h1, b0, W1, b1, W2, b2)


def _edge_mlp_body(g_ref, w1_ref, b1_ref, w2_ref, b2_ref, o_ref):
    t = jnp.tanh(g_ref[:, :64])
    h = jax.nn.relu(
        jnp.dot(t, w1_ref[...], preferred_element_type=jnp.float32) + b1_ref[...]
    )
    o_ref[...] = (
        jnp.dot(h, w2_ref[...], preferred_element_type=jnp.float32) + b2_ref[...]
    )


def _tc_edge_mlp(t0x, W1, b1, W2, b2, block_rows=8000):
    e = t0x.shape[0]
    f = 64
    grid = e // block_rows
    return pl.pallas_call(
        _edge_mlp_body,
        grid=(grid,),
        in_specs=[
            pl.BlockSpec((block_rows, 2 * f), lambda i: (i, 0)),
            pl.BlockSpec((f, f), lambda i: (0, 0)),
            pl.BlockSpec((f,), lambda i: (0,)),
            pl.BlockSpec((f, f), lambda i: (0, 0)),
            pl.BlockSpec((f,), lambda i: (0,)),
        ],
        out_specs=pl.BlockSpec((block_rows, f), lambda i: (i, 0)),
        out_shape=jax.ShapeDtypeStruct((e, f), jnp.float32),
    )(t0x, W1, b1, W2, b2)


def _final_dense_body(e0_ref, e1_ref, hw_ref, hb_ref, dw_ref, db_ref,
                      dd_ref, pre_ref):
    en = e0_ref[:, :64] + e1_ref[:, :64]
    dh = jnp.dot(en, hw_ref[...], preferred_element_type=jnp.float32) + hb_ref[...]
    dd_ref[...] = (
        jnp.dot(dh, dw_ref[...], preferred_element_type=jnp.float32) + db_ref[...]
    )
    half = dh.shape[1] // 2
    pre_ref[...] = jnp.concatenate([dh[:, half:], -dh[:, :half]], axis=1)


def _tc_final_dense(e0, e1, H_W, H_b, D_W, D_b, block_rows=2000):
    n = e0.shape[0]
    f = 64
    fo = H_W.shape[1]
    grid = n // block_rows
    return pl.pallas_call(
        _final_dense_body,
        grid=(grid,),
        in_specs=[
            pl.BlockSpec((block_rows, 2 * f), lambda i: (i, 0)),
            pl.BlockSpec((block_rows, 2 * f), lambda i: (i, 0)),
            pl.BlockSpec((f, fo), lambda i: (0, 0)),
            pl.BlockSpec((fo,), lambda i: (0,)),
            pl.BlockSpec((fo, fo), lambda i: (0, 0)),
            pl.BlockSpec((fo,), lambda i: (0,)),
        ],
        out_specs=[
            pl.BlockSpec((block_rows, fo), lambda i: (i, 0)),
            pl.BlockSpec((block_rows, fo), lambda i: (i, 0)),
        ],
        out_shape=[
            jax.ShapeDtypeStruct((n, fo), jnp.float32),
            jax.ShapeDtypeStruct((n, fo), jnp.float32),
        ],
    )(e0, e1, H_W, H_b, D_W, D_b)


def _combine_body(pre_ref, d0_ref, d1_ref, o_ref):
    o_ref[...] = pre_ref[...] - d0_ref[...] - d1_ref[...]


def _tc_combine(pre, d0, d1, block_rows=2000):
    n, f = pre.shape
    grid = n // block_rows
    return pl.pallas_call(
        _combine_body,
        grid=(grid,),
        in_specs=[pl.BlockSpec((block_rows, f), lambda i: (i, 0))] * 3,
        out_specs=pl.BlockSpec((block_rows, f), lambda i: (i, 0)),
        out_shape=jax.ShapeDtypeStruct((n, f), jnp.float32),
    )(pre, d0, d1)


def _sc_mesh():
    return plsc.VectorSubcoreMesh(
        core_axis_name="c", subcore_axis_name="s",
        num_cores=_NC, num_subcores=_NS)


def _seg_sum(table, src, dst, n):
    d = table.shape[1]
    e = src.shape[0]
    per_w = e // _NW
    nchunks = per_w // _CHUNK
    nzt = 10
    rows_per_tile = n // nzt
    zeros = jnp.zeros((rows_per_tile, d), jnp.float32)

    @functools.partial(
        pl.kernel,
        out_type=jax.ShapeDtypeStruct((_NC, n, d), jnp.float32),
        mesh=_sc_mesh(),
        scratch_types=[
            pltpu.VMEM((4, _CHUNK), jnp.int32),
            pltpu.VMEM((4, _CHUNK), jnp.int32),
            pltpu.VMEM((2, _CHUNK, d), jnp.float32),
            pltpu.VMEM_SHARED((n, d), jnp.float32),
        ] + [pltpu.SemaphoreType.DMA] * 12,
    )
    def seg_kernel(table_hbm, src_hbm, dst_hbm, zeros_hbm, out_hbm,
                   idxg_v, idxs_v, rows_v, acc_sh, *sems):
        c = lax.axis_index("c")
        s = lax.axis_index("s")
        wid = s * _NC + c
        base = wid * per_w
        gsem = sems[0:2]
        ssem = sems[2:4]
        igsem = sems[4:8]
        issem = sems[8:12]

        def ldg(ci, sl):
            return pltpu.make_async_copy(
                src_hbm.at[pl.ds(base + ci * _CHUNK, _CHUNK)],
                idxg_v.at[sl], igsem[sl])

        def lds(ci, sl):
            return pltpu.make_async_copy(
                dst_hbm.at[pl.ds(base + ci * _CHUNK, _CHUNK)],
                idxs_v.at[sl], issem[sl])

        def gather(sl, b):
            return pltpu.make_async_copy(
                table_hbm.at[idxg_v.at[sl]], rows_v.at[b], gsem[b])

        def scatter(sl, b):
            return pltpu.make_async_copy(
                rows_v.at[b], acc_sh.at[idxs_v.at[sl]], ssem[b])

        @pl.when(s < nzt)
        def _():
            pltpu.sync_copy(zeros_hbm,
                            acc_sh.at[pl.ds(s * rows_per_tile, rows_per_tile)])
        for u in range(4):
            ldg(u, u).start()
            lds(u, u).start()
        for b in range(2):
            ldg(b, b).wait()
            lds(b, b).wait()
            gather(b, b).start()
        plsc.subcore_barrier()

        @pl.loop(0, (nchunks + 3) // 4)
        def _(cq):
            for u in range(4):
                ci = cq * 4 + u
                b = u % 2
                ob = 1 - b
                osl = (u + 1) % 4
                psl = (u + 3) % 4

                @pl.when(ci < nchunks)
                def _():
                    gather(u, b).wait()
                    scatter(u, b).start(add=True)

                @pl.when(jnp.logical_and(ci >= 1, ci + 1 < nchunks))
                def _():
                    scatter(psl, ob).wait()
                    ldg(ci + 1, osl).wait()
                    lds(ci + 1, osl).wait()
                    gather(osl, ob).start()

                    @pl.when(ci + 3 < nchunks)
                    def _():
                        ldg(ci + 3, psl).start()
                        lds(ci + 3, psl).start()

        scatter((nchunks - 2) % 4, (nchunks - 2) % 2).wait()
        scatter((nchunks - 1) % 4, (nchunks - 1) % 2).wait()
        plsc.subcore_barrier()

        @pl.when(s < nzt)
        def _():
            pltpu.sync_copy(acc_sh.at[pl.ds(s * rows_per_tile, rows_per_tile)],
                            out_hbm.at[c, pl.ds(s * rows_per_tile, rows_per_tile)])

    out = seg_kernel(table, src, dst, zeros)
    return out[0], out[1]


def _seg_sum_and_edge_gather(ts, t2, src, dst, n):
    d = ts.shape[1]
    e = src.shape[0]
    per_w = e // _NW
    nchunks = per_w // _CHUNK
    nzt = 10
    rows_per_tile = n // nzt
    zeros = jnp.zeros((rows_per_tile, d), jnp.float32)

    @functools.partial(
        pl.kernel,
        out_type=(jax.ShapeDtypeStruct((_NC, n, d), jnp.float32),
                  jax.ShapeDtypeStruct((e, d), jnp.float32)),
        mesh=_sc_mesh(),
        scratch_types=[
            pltpu.VMEM((4, _CHUNK), jnp.int32),
            pltpu.VMEM((4, _CHUNK), jnp.int32),
            pltpu.VMEM((2, _CHUNK, d), jnp.float32),
            pltpu.VMEM((2, _CHUNK, d), jnp.float32),
            pltpu.VMEM_SHARED((n, d), jnp.float32),
        ] + [pltpu.SemaphoreType.DMA] * 16,
    )
    def fused_kernel(ts_hbm, t2_hbm, src_hbm, dst_hbm, zeros_hbm,
                     out_hbm, t0x_hbm,
                     idxg_v, idxs_v, r1_v, r2_v, acc_sh, *sems):
        c = lax.axis_index("c")
        s = lax.axis_index("s")
        wid = s * _NC + c
        base = wid * per_w
        g1sem = sems[0:2]
        g2sem = sems[2:4]
        ssem = sems[4:6]
        wsem = sems[6:8]
        igsem = sems[8:12]
        issem = sems[12:16]

        def ldg(ci, sl):
            return pltpu.make_async_copy(
                src_hbm.at[pl.ds(base + ci * _CHUNK, _CHUNK)],
                idxg_v.at[sl], igsem[sl])

        def lds(ci, sl):
            return pltpu.make_async_copy(
                dst_hbm.at[pl.ds(base + ci * _CHUNK, _CHUNK)],
                idxs_v.at[sl], issem[sl])

        def gather1(sl, b):
            return pltpu.make_async_copy(
                ts_hbm.at[idxg_v.at[sl]], r1_v.at[b], g1sem[b])

        def gather2(sl, b):
            return pltpu.make_async_copy(
                t2_hbm.at[idxs_v.at[sl]], r2_v.at[b], g2sem[b])

        def scatter(sl, b):
            return pltpu.make_async_copy(
                r1_v.at[b], acc_sh.at[idxs_v.at[sl]], ssem[b])

        def twrite(ci, b):
            return pltpu.make_async_copy(
                r2_v.at[b], t0x_hbm.at[pl.ds(base + ci * _CHUNK, _CHUNK)],
                wsem[b])

        @pl.when(s < nzt)
        def _():
            pltpu.sync_copy(zeros_hbm,
                            acc_sh.at[pl.ds(s * rows_per_tile, rows_per_tile)])
        for u in range(4):
            ldg(u, u).start()
            lds(u, u).start()
        for b in range(2):
            ldg(b, b).wait()
            lds(b, b).wait()
            gather1(b, b).start()
            gather2(b, b).start()
        plsc.subcore_barrier()

        @pl.loop(0, (nchunks + 3) // 4)
        def _(cq):
            for u in range(4):
                ci = cq * 4 + u
                b = u % 2
                ob = 1 - b
                osl = (u + 1) % 4
                psl = (u + 3) % 4

                @pl.when(ci < nchunks)
                def _():
                    gather1(u, b).wait()
                    scatter(u, b).start(add=True)
                    gather2(u, b).wait()

                @pl.when(jnp.logical_and(ci >= 1, ci + 1 < nchunks))
                def _():
                    scatter(psl, ob).wait()
                    twrite(ci - 1, ob).wait()
                    ldg(ci + 1, osl).wait()
                    lds(ci + 1, osl).wait()
                    gather1(osl, ob).start()
                    gather2(osl, ob).start()

                    @pl.when(ci + 3 < nchunks)
                    def _():
                        ldg(ci + 3, psl).start()
                        lds(ci + 3, psl).start()

                @pl.when(ci < nchunks)
                def _():
                    @pl.loop(0, _CHUNK, unroll=8)
                    def _(ri):
                        for j in range(d // 2 // 16):
                            lo = pl.ds(j * 16, 16)
                            hi = pl.ds(d // 2 + j * 16, 16)
                            r2_v[b, ri, lo] = r1_v[b, ri, hi] + r2_v[b, ri, lo]

                    twrite(ci, b).start()

        for ci in (nchunks - 2, nchunks - 1):
            b = ci % 2
            scatter(ci % 4, b).wait()
            twrite(ci, b).wait()
        plsc.subcore_barrier()

        @pl.when(s < nzt)
        def _():
            pltpu.sync_copy(acc_sh.at[pl.ds(s * rows_per_tile, rows_per_tile)],
                            out_hbm.at[c, pl.ds(s * rows_per_tile, rows_per_tile)])

    return fused_kernel(ts, t2, src, dst, zeros)


def _gather_mul_seg_sum(kpad, p, src, dst, n):
    d = kpad.shape[1]
    dp = p.shape[1]
    e = src.shape[0]
    per_w = e // _NW
    nchunks = per_w // _CHUNK
    nzt = 10
    rows_per_tile = n // nzt
    zeros = jnp.zeros((rows_per_tile, d), jnp.float32)

    @functools.partial(
        pl.kernel,
        out_type=jax.ShapeDtypeStruct((_NC, n, d), jnp.float32),
        mesh=_sc_mesh(),
        scratch_types=[
            pltpu.VMEM((4, _CHUNK), jnp.int32),
            pltpu.VMEM((4, _CHUNK), jnp.int32),
            pltpu.VMEM((2, _CHUNK, d), jnp.float32),
            pltpu.VMEM((2, _CHUNK, dp), jnp.float32),
            pltpu.VMEM_SHARED((n, d), jnp.float32),
        ] + [pltpu.SemaphoreType.DMA] * 14,
    )
    def gm_kernel(k_hbm, p_hbm, src_hbm, dst_hbm, zeros_hbm, out_hbm,
                  idxg_v, idxs_v, rows_v, pv_v, acc_sh, *sems):
        c = lax.axis_index("c")
        s = lax.axis_index("s")
        wid = s * _NC + c
        base = wid * per_w
        ksem = sems[0:2]
        psem = sems[2:4]
        ssem = sems[4:6]
        igsem = sems[6:10]
        issem = sems[10:14]

        def ldg(ci, sl):
            return pltpu.make_async_copy(
                src_hbm.at[pl.ds(base + ci * _CHUNK, _CHUNK)],
                idxg_v.at[sl], igsem[sl])

        def lds(ci, sl):
            return pltpu.make_async_copy(
                dst_hbm.at[pl.ds(base + ci * _CHUNK, _CHUNK)],
                idxs_v.at[sl], issem[sl])

        def gather(sl, b):
            return pltpu.make_async_copy(
                k_hbm.at[idxg_v.at[sl]], rows_v.at[b], ksem[b])

        def pread(ci, b):
            return pltpu.make_async_copy(
                p_hbm.at[pl.ds(base + ci * _CHUNK, _CHUNK)], pv_v.at[b],
                psem[b])

        def scatter(sl, b):
            return pltpu.make_async_copy(
                rows_v.at[b], acc_sh.at[idxs_v.at[sl]], ssem[b])

        @pl.when(s < nzt)
        def _():
            pltpu.sync_copy(zeros_hbm,
                            acc_sh.at[pl.ds(s * rows_per_tile, rows_per_tile)])
        for u in range(4):
            ldg(u, u).start()
            lds(u, u).start()
        for b in range(2):
            ldg(b, b).wait()
            lds(b, b).wait()
            gather(b, b).start()
            pread(b, b).start()
        plsc.subcore_barrier()

        @pl.loop(0, (nchunks + 3) // 4)
        def _(cq):
            for u in range(4):
                ci = cq * 4 + u
                b = u % 2
                ob = 1 - b
                osl = (u + 1) % 4
                psl = (u + 3) % 4

                @pl.when(ci < nchunks)
                def _():
                    gather(u, b).wait()
                    pread(ci, b).wait()

                @pl.when(jnp.logical_and(ci >= 1, ci + 1 < nchunks))
                def _():
                    scatter(psl, ob).wait()
                    ldg(ci + 1, osl).wait()
                    lds(ci + 1, osl).wait()
                    gather(osl, ob).start()
                    pread(ci + 1, ob).start()

                    @pl.when(ci + 3 < nchunks)
                    def _():
                        ldg(ci + 3, psl).start()
                        lds(ci + 3, psl).start()

                @pl.when(ci < nchunks)
                def _():
                    @pl.loop(0, _CHUNK, unroll=8)
                    def _(ri):
                        for j in range(dp // 16):
                            sl = pl.ds(j * 16, 16)
                            rows_v[b, ri, sl] = (
                                rows_v[b, ri, sl] * pv_v[b, ri, sl])

                    scatter(u, b).start(add=True)

        scatter((nchunks - 2) % 4, (nchunks - 2) % 2).wait()
        scatter((nchunks - 1) % 4, (nchunks - 1) % 2).wait()
        plsc.subcore_barrier()

        @pl.when(s < nzt)
        def _():
            pltpu.sync_copy(acc_sh.at[pl.ds(s * rows_per_tile, rows_per_tile)],
                            out_hbm.at[c, pl.ds(s * rows_per_tile, rows_per_tile)])

    out = gm_kernel(kpad, p, src, dst, zeros)
    return out[0], out[1]


def kernel(x, src, dst, encK_W, encK_b, encP1_W, encP1_b, encP2_W, encP2_b,
           Kf0_W, Kf0_b, Kf1_W, Kf1_b, Kf2_W, Kf2_b,
           Uf0_W, Uf0_b, Uf1_W, Uf1_b, Uf2_W, Uf2_b, H_W, H_b, D_W, D_b):
    n, in_dim = x.shape

    WB = encK_W @ Kf0_W
    bB = encK_b @ Kf0_W
    W1 = encP1_W @ Uf0_W
    b1 = encP1_b @ Uf0_W + Uf0_b
    W2 = encP2_W @ Uf0_W
    b2 = encP2_b @ Uf0_W
    Wcat = jnp.concatenate(
        [WB, W1, W2, jnp.zeros_like(W2)], axis=1)
    bcat = jnp.concatenate([bB, b1, b2, jnp.zeros_like(b2)], axis=0)

    pre_all = _tc_mm_bias(x, Wcat, bcat)
    ts = lax.slice(pre_all, (0, 0), (n, 128))
    t2 = lax.slice(pre_all, (0, 128), (n, 256))

    part, t0x = _seg_sum_and_edge_gather(ts, t2, src, dst, n)
    kpad = _tc_node_mlp(part[0], part[1],
                        Kf0_b, Kf1_W, Kf1_b, Kf2_W, Kf2_b)

    P = _tc_edge_mlp(t0x, Uf1_W, Uf1_b, Uf2_W, Uf2_b)

    e0, e1 = _gather_mul_seg_sum(kpad, P, src, dst, n)
    dd, pre = _tc_final_dense(e0, e1, H_W, H_b, D_W, D_b)

    d0, d1 = _seg_sum(dd, src, dst, n)
    return _tc_combine(pre, d0, d1)


# speedup vs baseline: 1.3653x; 1.0712x over previous
import functools

import jax
import jax.numpy as jnp
from jax import lax
from jax.experimental import pallas as pl
from jax.experimental.pallas import tpu as pltpu
from jax.experimental.pallas import tpu_sc as plsc

_NC, _NS = 2, 16
_NW = _NC * _NS
_CHUNK = 80
_DCHUNK = 40


def _mm_bias_body(x_ref, w_ref, b_ref, o_ref):
    o_ref[...] = (
        jnp.dot(x_ref[...], w_ref[...], preferred_element_type=jnp.float32)
        + b_ref[...]
    )


def _tc_mm_bias(x, W, b, block_rows=2000):
    n, fin = x.shape
    fout = W.shape[1]
    grid = n // block_rows
    return pl.pallas_call(
        _mm_bias_body,
        grid=(grid,),
        in_specs=[
            pl.BlockSpec((block_rows, fin), lambda i: (i, 0)),
            pl.BlockSpec((fin, fout), lambda i: (0, 0)),
            pl.BlockSpec((fout,), lambda i: (0,)),
        ],
        out_specs=pl.BlockSpec((block_rows, fout), lambda i: (i, 0)),
        out_shape=jax.ShapeDtypeStruct((n, fout), jnp.float32),
    )(x, W, b)


def _kmlp_body(h0_ref, h1_ref, b0_ref, w1_ref, b1_ref, w2_ref, b2_ref, o_ref):
    h = jnp.tanh(h0_ref[:, :64] + h1_ref[:, :64] + b0_ref[...])
    h = jax.nn.relu(
        jnp.dot(h, w1_ref[...], preferred_element_type=jnp.float32) + b1_ref[...]
    )
    k = jnp.dot(h, w2_ref[...], preferred_element_type=jnp.float32) + b2_ref[...]
    o_ref[...] = jnp.concatenate([k, jnp.zeros_like(k)], axis=1)


def _tc_node_mlp(h0, h1, b0, W1, b1, W2, b2, block_rows=2000):
    n = h0.shape[0]
    f = 64
    grid = n // block_rows
    return pl.pallas_call(
        _kmlp_body,
        grid=(grid,),
        in_specs=[
            pl.BlockSpec((block_rows, 2 * f), lambda i: (i, 0)),
            pl.BlockSpec((block_rows, 2 * f), lambda i: (i, 0)),
            pl.BlockSpec((f,), lambda i: (0,)),
            pl.BlockSpec((f, f), lambda i: (0, 0)),
            pl.BlockSpec((f,), lambda i: (0,)),
            pl.BlockSpec((f, f), lambda i: (0, 0)),
            pl.BlockSpec((f,), lambda i: (0,)),
        ],
        out_specs=pl.BlockSpec((block_rows, 2 * f), lambda i: (i, 0)),
        out_shape=jax.ShapeDtypeStruct((n, 2 * f), jnp.float32),
    )(h0, h1, b0, W1, b1, W2, b2)


def _edge_mlp_body(g_ref, w1_ref, b1_ref, w2_ref, b2_ref, o_ref):
    t = jnp.tanh(g_ref[:, :64])
    h = jax.nn.relu(
        jnp.dot(t, w1_ref[...], preferred_element_type=jnp.float32) + b1_ref[...]
    )
    o_ref[...] = (
        jnp.dot(h, w2_ref[...], preferred_element_type=jnp.float32) + b2_ref[...]
    )


def _tc_edge_mlp(t0x, W1, b1, W2, b2, block_rows=8000):
    e = t0x.shape[0]
    f = 64
    grid = e // block_rows
    return pl.pallas_call(
        _edge_mlp_body,
        grid=(grid,),
        in_specs=[
            pl.BlockSpec((block_rows, 2 * f), lambda i: (i, 0)),
            pl.BlockSpec((f, f), lambda i: (0, 0)),
            pl.BlockSpec((f,), lambda i: (0,)),
            pl.BlockSpec((f, f), lambda i: (0, 0)),
            pl.BlockSpec((f,), lambda i: (0,)),
        ],
        out_specs=pl.BlockSpec((block_rows, f), lambda i: (i, 0)),
        out_shape=jax.ShapeDtypeStruct((e, f), jnp.float32),
    )(t0x, W1, b1, W2, b2)


def _final_dense_body(e0_ref, e1_ref, hw_ref, hb_ref, dw_ref, db_ref,
                      dd_ref, pre_ref):
    en = e0_ref[:, :64] + e1_ref[:, :64]
    dh = jnp.dot(en, hw_ref[...], preferred_element_type=jnp.float32) + hb_ref[...]
    dd_ref[...] = (
        jnp.dot(dh, dw_ref[...], preferred_element_type=jnp.float32) + db_ref[...]
    )
    half = dh.shape[1] // 2
    pre_ref[...] = jnp.concatenate([dh[:, half:], -dh[:, :half]], axis=1)


def _tc_final_dense(e0, e1, H_W, H_b, D_W, D_b, block_rows=2000):
    n = e0.shape[0]
    f = 64
    fo = H_W.shape[1]
    grid = n // block_rows
    return pl.pallas_call(
        _final_dense_body,
        grid=(grid,),
        in_specs=[
            pl.BlockSpec((block_rows, 2 * f), lambda i: (i, 0)),
            pl.BlockSpec((block_rows, 2 * f), lambda i: (i, 0)),
            pl.BlockSpec((f, fo), lambda i: (0, 0)),
            pl.BlockSpec((fo,), lambda i: (0,)),
            pl.BlockSpec((fo, fo), lambda i: (0, 0)),
            pl.BlockSpec((fo,), lambda i: (0,)),
        ],
        out_specs=[
            pl.BlockSpec((block_rows, fo), lambda i: (i, 0)),
            pl.BlockSpec((block_rows, fo), lambda i: (i, 0)),
        ],
        out_shape=[
            jax.ShapeDtypeStruct((n, fo), jnp.float32),
            jax.ShapeDtypeStruct((n, fo), jnp.float32),
        ],
    )(e0, e1, H_W, H_b, D_W, D_b)


def _combine_body(pre_ref, d0_ref, d1_ref, o_ref):
    o_ref[...] = pre_ref[...] - d0_ref[...] - d1_ref[...]


def _tc_combine(pre, d0, d1, block_rows=2000):
    n, f = pre.shape
    grid = n // block_rows
    return pl.pallas_call(
        _combine_body,
        grid=(grid,),
        in_specs=[pl.BlockSpec((block_rows, f), lambda i: (i, 0))] * 3,
        out_specs=pl.BlockSpec((block_rows, f), lambda i: (i, 0)),
        out_shape=jax.ShapeDtypeStruct((n, f), jnp.float32),
    )(pre, d0, d1)


def _sc_mesh():
    return plsc.VectorSubcoreMesh(
        core_axis_name="c", subcore_axis_name="s",
        num_cores=_NC, num_subcores=_NS)


def _seg_sum(table, src, dst, n):
    d = table.shape[1]
    e = src.shape[0]
    per_w = e // _NW
    nchunks = per_w // _DCHUNK
    nzt = 10
    rows_per_tile = n // nzt
    zeros = jnp.zeros((rows_per_tile, d), jnp.float32)

    @functools.partial(
        pl.kernel,
        out_type=jax.ShapeDtypeStruct((_NC, n, d), jnp.float32),
        mesh=_sc_mesh(),
        scratch_types=[
            pltpu.VMEM((8, _DCHUNK), jnp.int32),
            pltpu.VMEM((8, _DCHUNK), jnp.int32),
            pltpu.VMEM((4, _DCHUNK, d), jnp.float32),
            pltpu.VMEM_SHARED((n, d), jnp.float32),
        ] + [pltpu.SemaphoreType.DMA] * 24,
    )
    def seg_kernel(table_hbm, src_hbm, dst_hbm, zeros_hbm, out_hbm,
                   idxg_v, idxs_v, rows_v, acc_sh, *sems):
        c = lax.axis_index("c")
        s = lax.axis_index("s")
        wid = s * _NC + c
        base = wid * per_w
        gsem = sems[0:4]
        ssem = sems[4:8]
        igsem = sems[8:16]
        issem = sems[16:24]

        def ldg(ci, sl):
            return pltpu.make_async_copy(
                src_hbm.at[pl.ds(base + ci * _DCHUNK, _DCHUNK)],
                idxg_v.at[sl], igsem[sl])

        def lds(ci, sl):
            return pltpu.make_async_copy(
                dst_hbm.at[pl.ds(base + ci * _DCHUNK, _DCHUNK)],
                idxs_v.at[sl], issem[sl])

        def gather(sl, b):
            return pltpu.make_async_copy(
                table_hbm.at[idxg_v.at[sl]], rows_v.at[b], gsem[b])

        def scatter(sl, b):
            return pltpu.make_async_copy(
                rows_v.at[b], acc_sh.at[idxs_v.at[sl]], ssem[b])

        @pl.when(s < nzt)
        def _():
            pltpu.sync_copy(zeros_hbm,
                            acc_sh.at[pl.ds(s * rows_per_tile, rows_per_tile)])
        for u in range(6):
            ldg(u, u).start()
            lds(u, u).start()
        for b in range(2):
            ldg(b, b).wait()
            lds(b, b).wait()
            gather(b, b).start()
        plsc.subcore_barrier()

        @pl.loop(0, (nchunks + 7) // 8)
        def _(cq):
            for u in range(8):
                ci = cq * 8 + u
                rb = u % 4
                nrb = (u + 2) % 4
                nsl = (u + 2) % 8
                psl = (u + 6) % 8

                @pl.when(ci < nchunks)
                def _():
                    gather(u, rb).wait()
                    scatter(u, rb).start(add=True)

                @pl.when(ci + 2 < nchunks)
                def _():
                    @pl.when(ci >= 2)
                    def _():
                        scatter(psl, nrb).wait()
                    ldg(ci + 2, nsl).wait()
                    lds(ci + 2, nsl).wait()
                    gather(nsl, nrb).start()

                    @pl.when(ci + 6 < nchunks)
                    def _():
                        ldg(ci + 6, psl).start()
                        lds(ci + 6, psl).start()

        for ci in range(nchunks - 4, nchunks):
            scatter(ci % 8, ci % 4).wait()
        plsc.subcore_barrier()

        @pl.when(s < nzt)
        def _():
            pltpu.sync_copy(acc_sh.at[pl.ds(s * rows_per_tile, rows_per_tile)],
                            out_hbm.at[c, pl.ds(s * rows_per_tile, rows_per_tile)])

    out = seg_kernel(table, src, dst, zeros)
    return out[0], out[1]


def _seg_sum_and_edge_gather(ts, t2, src, dst, n):
    d = ts.shape[1]
    e = src.shape[0]
    per_w = e // _NW
    nchunks = per_w // _CHUNK
    nzt = 10
    rows_per_tile = n // nzt
    zeros = jnp.zeros((rows_per_tile, d), jnp.float32)

    @functools.partial(
        pl.kernel,
        out_type=(jax.ShapeDtypeStruct((_NC, n, d), jnp.float32),
                  jax.ShapeDtypeStruct((e, d), jnp.float32)),
        mesh=_sc_mesh(),
        scratch_types=[
            pltpu.VMEM((4, _CHUNK), jnp.int32),
            pltpu.VMEM((4, _CHUNK), jnp.int32),
            pltpu.VMEM((2, _CHUNK, d), jnp.float32),
            pltpu.VMEM((2, _CHUNK, d), jnp.float32),
            pltpu.VMEM_SHARED((n, d), jnp.float32),
        ] + [pltpu.SemaphoreType.DMA] * 16,
    )
    def fused_kernel(ts_hbm, t2_hbm, src_hbm, dst_hbm, zeros_hbm,
                     out_hbm, t0x_hbm,
                     idxg_v, idxs_v, r1_v, r2_v, acc_sh, *sems):
        c = lax.axis_index("c")
        s = lax.axis_index("s")
        wid = s * _NC + c
        base = wid * per_w
        g1sem = sems[0:2]
        g2sem = sems[2:4]
        ssem = sems[4:6]
        wsem = sems[6:8]
        igsem = sems[8:12]
        issem = sems[12:16]

        def ldg(ci, sl):
            return pltpu.make_async_copy(
                src_hbm.at[pl.ds(base + ci * _CHUNK, _CHUNK)],
                idxg_v.at[sl], igsem[sl])

        def lds(ci, sl):
            return pltpu.make_async_copy(
                dst_hbm.at[pl.ds(base + ci * _CHUNK, _CHUNK)],
                idxs_v.at[sl], issem[sl])

        def gather1(sl, b):
            return pltpu.make_async_copy(
                ts_hbm.at[idxg_v.at[sl]], r1_v.at[b], g1sem[b])

        def gather2(sl, b):
            return pltpu.make_async_copy(
                t2_hbm.at[idxs_v.at[sl]], r2_v.at[b], g2sem[b])

        def scatter(sl, b):
            return pltpu.make_async_copy(
                r1_v.at[b], acc_sh.at[idxs_v.at[sl]], ssem[b])

        def twrite(ci, b):
            return pltpu.make_async_copy(
                r2_v.at[b], t0x_hbm.at[pl.ds(base + ci * _CHUNK, _CHUNK)],
                wsem[b])

        @pl.when(s < nzt)
        def _():
            pltpu.sync_copy(zeros_hbm,
                            acc_sh.at[pl.ds(s * rows_per_tile, rows_per_tile)])
        for u in range(4):
            ldg(u, u).start()
            lds(u, u).start()
        for b in range(2):
            ldg(b, b).wait()
            lds(b, b).wait()
            gather1(b, b).start()
            gather2(b, b).start()
        plsc.subcore_barrier()

        @pl.loop(0, (nchunks + 3) // 4)
        def _(cq):
            for u in range(4):
                ci = cq * 4 + u
                b = u % 2
                ob = 1 - b
                osl = (u + 1) % 4
                psl = (u + 3) % 4

                @pl.when(ci < nchunks)
                def _():
                    gather1(u, b).wait()
                    scatter(u, b).start(add=True)
                    gather2(u, b).wait()

                @pl.when(jnp.logical_and(ci >= 1, ci + 1 < nchunks))
                def _():
                    scatter(psl, ob).wait()
                    twrite(ci - 1, ob).wait()
                    ldg(ci + 1, osl).wait()
                    lds(ci + 1, osl).wait()
                    gather1(osl, ob).start()
                    gather2(osl, ob).start()

                    @pl.when(ci + 3 < nchunks)
                    def _():
                        ldg(ci + 3, psl).start()
                        lds(ci + 3, psl).start()

                @pl.when(ci < nchunks)
                def _():
                    @pl.loop(0, _CHUNK, unroll=8)
                    def _(ri):
                        for j in range(d // 2 // 16):
                            lo = pl.ds(j * 16, 16)
                            hi = pl.ds(d // 2 + j * 16, 16)
                            r2_v[b, ri, lo] = r1_v[b, ri, hi] + r2_v[b, ri, lo]

                    twrite(ci, b).start()

        for ci in (nchunks - 2, nchunks - 1):
            b = ci % 2
            scatter(ci % 4, b).wait()
            twrite(ci, b).wait()
        plsc.subcore_barrier()

        @pl.when(s < nzt)
        def _():
            pltpu.sync_copy(acc_sh.at[pl.ds(s * rows_per_tile, rows_per_tile)],
                            out_hbm.at[c, pl.ds(s * rows_per_tile, rows_per_tile)])

    return fused_kernel(ts, t2, src, dst, zeros)


def _gather_mul_seg_sum(kpad, p, src, dst, n):
    d = kpad.shape[1]
    dp = p.shape[1]
    e = src.shape[0]
    per_w = e // _NW
    nchunks = per_w // _DCHUNK
    nzt = 10
    rows_per_tile = n // nzt
    zeros = jnp.zeros((rows_per_tile, d), jnp.float32)

    @functools.partial(
        pl.kernel,
        out_type=jax.ShapeDtypeStruct((_NC, n, d), jnp.float32),
        mesh=_sc_mesh(),
        scratch_types=[
            pltpu.VMEM((8, _DCHUNK), jnp.int32),
            pltpu.VMEM((8, _DCHUNK), jnp.int32),
            pltpu.VMEM((4, _DCHUNK, d), jnp.float32),
            pltpu.VMEM((4, _DCHUNK, dp), jnp.float32),
            pltpu.VMEM_SHARED((n, d), jnp.float32),
        ] + [pltpu.SemaphoreType.DMA] * 28,
    )
    def gm_kernel(k_hbm, p_hbm, src_hbm, dst_hbm, zeros_hbm, out_hbm,
                  idxg_v, idxs_v, rows_v, pv_v, acc_sh, *sems):
        c = lax.axis_index("c")
        s = lax.axis_index("s")
        wid = s * _NC + c
        base = wid * per_w
        ksem = sems[0:4]
        psem = sems[4:8]
        ssem = sems[8:12]
        igsem = sems[12:20]
        issem = sems[20:28]

        def ldg(ci, sl):
            return pltpu.make_async_copy(
                src_hbm.at[pl.ds(base + ci * _DCHUNK, _DCHUNK)],
                idxg_v.at[sl], igsem[sl])

        def lds(ci, sl):
            return pltpu.make_async_copy(
                dst_hbm.at[pl.ds(base + ci * _DCHUNK, _DCHUNK)],
                idxs_v.at[sl], issem[sl])

        def gather(sl, b):
            return pltpu.make_async_copy(
                k_hbm.at[idxg_v.at[sl]], rows_v.at[b], ksem[b])

        def pread(ci, b):
            return pltpu.make_async_copy(
                p_hbm.at[pl.ds(base + ci * _DCHUNK, _DCHUNK)], pv_v.at[b],
                psem[b])

        def scatter(sl, b):
            return pltpu.make_async_copy(
                rows_v.at[b], acc_sh.at[idxs_v.at[sl]], ssem[b])

        @pl.when(s < nzt)
        def _():
            pltpu.sync_copy(zeros_hbm,
                            acc_sh.at[pl.ds(s * rows_per_tile, rows_per_tile)])
        for u in range(6):
            ldg(u, u).start()
            lds(u, u).start()
        for b in range(2):
            ldg(b, b).wait()
            lds(b, b).wait()
            gather(b, b).start()
            pread(b, b).start()
        plsc.subcore_barrier()

        @pl.loop(0, (nchunks + 7) // 8)
        def _(cq):
            for u in range(8):
                ci = cq * 8 + u
                rb = u % 4
                nrb = (u + 2) % 4
                nsl = (u + 2) % 8
                psl = (u + 6) % 8

                @pl.when(ci < nchunks)
                def _():
                    gather(u, rb).wait()
                    pread(ci, rb).wait()

                @pl.when(ci + 2 < nchunks)
                def _():
                    @pl.when(ci >= 2)
                    def _():
                        scatter(psl, nrb).wait()
                    ldg(ci + 2, nsl).wait()
                    lds(ci + 2, nsl).wait()
                    gather(nsl, nrb).start()
                    pread(ci + 2, nrb).start()

                    @pl.when(ci + 6 < nchunks)
                    def _():
                        ldg(ci + 6, psl).start()
                        lds(ci + 6, psl).start()

                @pl.when(ci < nchunks)
                def _():
                    @pl.loop(0, _DCHUNK, unroll=8)
                    def _(ri):
                        for j in range(dp // 16):
                            sl = pl.ds(j * 16, 16)
                            rows_v[rb, ri, sl] = (
                                rows_v[rb, ri, sl] * pv_v[rb, ri, sl])

                    scatter(u, rb).start(add=True)

        for ci in range(nchunks - 4, nchunks):
            scatter(ci % 8, ci % 4).wait()
        plsc.subcore_barrier()

        @pl.when(s < nzt)
        def _():
            pltpu.sync_copy(acc_sh.at[pl.ds(s * rows_per_tile, rows_per_tile)],
                            out_hbm.at[c, pl.ds(s * rows_per_tile, rows_per_tile)])

    out = gm_kernel(kpad, p, src, dst, zeros)
    return out[0], out[1]


def kernel(x, src, dst, encK_W, encK_b, encP1_W, encP1_b, encP2_W, encP2_b,
           Kf0_W, Kf0_b, Kf1_W, Kf1_b, Kf2_W, Kf2_b,
           Uf0_W, Uf0_b, Uf1_W, Uf1_b, Uf2_W, Uf2_b, H_W, H_b, D_W, D_b):
    n, in_dim = x.shape

    WB = encK_W @ Kf0_W
    bB = encK_b @ Kf0_W
    W1 = encP1_W @ Uf0_W
    b1 = encP1_b @ Uf0_W + Uf0_b
    W2 = encP2_W @ Uf0_W
    b2 = encP2_b @ Uf0_W
    Wcat = jnp.concatenate(
        [WB, W1, W2, jnp.zeros_like(W2)], axis=1)
    bcat = jnp.concatenate([bB, b1, b2, jnp.zeros_like(b2)], axis=0)

    pre_all = _tc_mm_bias(x, Wcat, bcat)
    ts = lax.slice(pre_all, (0, 0), (n, 128))
    t2 = lax.slice(pre_all, (0, 128), (n, 256))

    part, t0x = _seg_sum_and_edge_gather(ts, t2, src, dst, n)
    kpad = _tc_node_mlp(part[0], part[1],
                        Kf0_b, Kf1_W, Kf1_b, Kf2_W, Kf2_b)

    P = _tc_edge_mlp(t0x, Uf1_W, Uf1_b, Uf2_W, Uf2_b)

    e0, e1 = _gather_mul_seg_sum(kpad, P, src, dst, n)
    dd, pre = _tc_final_dense(e0, e1, H_W, H_b, D_W, D_b)

    d0, d1 = _seg_sum(dd, src, dst, n)
    return _tc_combine(pre, d0, d1)


# speedup vs baseline: 1.4081x; 1.0313x over previous
import functools

import jax
import jax.numpy as jnp
from jax import lax
from jax.experimental import pallas as pl
from jax.experimental.pallas import tpu as pltpu
from jax.experimental.pallas import tpu_sc as plsc

_NC, _NS = 2, 16
_NW = _NC * _NS
_CHUNK = 80
_DCHUNK = 40


def _mm_bias_body(x_ref, w_ref, b_ref, o_ref):
    o_ref[...] = (
        jnp.dot(x_ref[...], w_ref[...], preferred_element_type=jnp.float32)
        + b_ref[...]
    )


def _tc_mm_bias(x, W, b, block_rows=2000):
    n, fin = x.shape
    fout = W.shape[1]
    grid = n // block_rows
    return pl.pallas_call(
        _mm_bias_body,
        grid=(grid,),
        in_specs=[
            pl.BlockSpec((block_rows, fin), lambda i: (i, 0)),
            pl.BlockSpec((fin, fout), lambda i: (0, 0)),
            pl.BlockSpec((fout,), lambda i: (0,)),
        ],
        out_specs=pl.BlockSpec((block_rows, fout), lambda i: (i, 0)),
        out_shape=jax.ShapeDtypeStruct((n, fout), jnp.float32),
    )(x, W, b)


def _kmlp_body(h0_ref, h1_ref, b0_ref, w1_ref, b1_ref, w2_ref, b2_ref, o_ref):
    h = jnp.tanh(h0_ref[:, :64] + h1_ref[:, :64] + b0_ref[...])
    h = jax.nn.relu(
        jnp.dot(h, w1_ref[...], preferred_element_type=jnp.float32) + b1_ref[...]
    )
    k = jnp.dot(h, w2_ref[...], preferred_element_type=jnp.float32) + b2_ref[...]
    o_ref[...] = jnp.concatenate([k, jnp.zeros_like(k)], axis=1)


def _tc_node_mlp(h0, h1, b0, W1, b1, W2, b2, block_rows=2000):
    n = h0.shape[0]
    f = 64
    grid = n // block_rows
    return pl.pallas_call(
        _kmlp_body,
        grid=(grid,),
        in_specs=[
            pl.BlockSpec((block_rows, 2 * f), lambda i: (i, 0)),
            pl.BlockSpec((block_rows, 2 * f), lambda i: (i, 0)),
            pl.BlockSpec((f,), lambda i: (0,)),
            pl.BlockSpec((f, f), lambda i: (0, 0)),
            pl.BlockSpec((f,), lambda i: (0,)),
            pl.BlockSpec((f, f), lambda i: (0, 0)),
            pl.BlockSpec((f,), lambda i: (0,)),
        ],
        out_specs=pl.BlockSpec((block_rows, 2 * f), lambda i: (i, 0)),
        out_shape=jax.ShapeDtypeStruct((n, 2 * f), jnp.float32),
    )(h0, h1, b0, W1, b1, W2, b2)


def _edge_mlp_body(g_ref, w1_ref, b1_ref, w2_ref, b2_ref, o_ref):
    t = jnp.tanh(g_ref[:, :64])
    h = jax.nn.relu(
        jnp.dot(t, w1_ref[...], preferred_element_type=jnp.float32) + b1_ref[...]
    )
    o_ref[...] = (
        jnp.dot(h, w2_ref[...], preferred_element_type=jnp.float32) + b2_ref[...]
    )


def _tc_edge_mlp(t0x, W1, b1, W2, b2, block_rows=8000):
    e = t0x.shape[0]
    f = 64
    grid = e // block_rows
    return pl.pallas_call(
        _edge_mlp_body,
        grid=(grid,),
        in_specs=[
            pl.BlockSpec((block_rows, 2 * f), lambda i: (i, 0)),
            pl.BlockSpec((f, f), lambda i: (0, 0)),
            pl.BlockSpec((f,), lambda i: (0,)),
            pl.BlockSpec((f, f), lambda i: (0, 0)),
            pl.BlockSpec((f,), lambda i: (0,)),
        ],
        out_specs=pl.BlockSpec((block_rows, f), lambda i: (i, 0)),
        out_shape=jax.ShapeDtypeStruct((e, f), jnp.float32),
    )(t0x, W1, b1, W2, b2)


def _final_dense_body(e0_ref, e1_ref, hw_ref, hb_ref, dw_ref, db_ref,
                      dd_ref, pre_ref):
    en = e0_ref[:, :64] + e1_ref[:, :64]
    dh = jnp.dot(en, hw_ref[...], preferred_element_type=jnp.float32) + hb_ref[...]
    dd_ref[...] = (
        jnp.dot(dh, dw_ref[...], preferred_element_type=jnp.float32) + db_ref[...]
    )
    half = dh.shape[1] // 2
    pre_ref[...] = jnp.concatenate([dh[:, half:], -dh[:, :half]], axis=1)


def _tc_final_dense(e0, e1, H_W, H_b, D_W, D_b, block_rows=2000):
    n = e0.shape[0]
    f = 64
    fo = H_W.shape[1]
    grid = n // block_rows
    return pl.pallas_call(
        _final_dense_body,
        grid=(grid,),
        in_specs=[
            pl.BlockSpec((block_rows, 2 * f), lambda i: (i, 0)),
            pl.BlockSpec((block_rows, 2 * f), lambda i: (i, 0)),
            pl.BlockSpec((f, fo), lambda i: (0, 0)),
            pl.BlockSpec((fo,), lambda i: (0,)),
            pl.BlockSpec((fo, fo), lambda i: (0, 0)),
            pl.BlockSpec((fo,), lambda i: (0,)),
        ],
        out_specs=[
            pl.BlockSpec((block_rows, fo), lambda i: (i, 0)),
            pl.BlockSpec((block_rows, fo), lambda i: (i, 0)),
        ],
        out_shape=[
            jax.ShapeDtypeStruct((n, fo), jnp.float32),
            jax.ShapeDtypeStruct((n, fo), jnp.float32),
        ],
    )(e0, e1, H_W, H_b, D_W, D_b)


def _combine_body(pre_ref, d0_ref, d1_ref, o_ref):
    o_ref[...] = pre_ref[...] - d0_ref[...] - d1_ref[...]


def _tc_combine(pre, d0, d1, block_rows=2000):
    n, f = pre.shape
    grid = n // block_rows
    return pl.pallas_call(
        _combine_body,
        grid=(grid,),
        in_specs=[pl.BlockSpec((block_rows, f), lambda i: (i, 0))] * 3,
        out_specs=pl.BlockSpec((block_rows, f), lambda i: (i, 0)),
        out_shape=jax.ShapeDtypeStruct((n, f), jnp.float32),
    )(pre, d0, d1)


def _sc_mesh():
    return plsc.VectorSubcoreMesh(
        core_axis_name="c", subcore_axis_name="s",
        num_cores=_NC, num_subcores=_NS)


def _seg_sum(table, src, dst, n):
    ck = _CHUNK
    d = table.shape[1]
    e = src.shape[0]
    per_w = e // _NW
    nchunks = per_w // ck
    nzt = 10
    rows_per_tile = n // nzt
    zeros = jnp.zeros((rows_per_tile, d), jnp.float32)

    @functools.partial(
        pl.kernel,
        out_type=jax.ShapeDtypeStruct((_NC, n, d), jnp.float32),
        mesh=_sc_mesh(),
        scratch_types=[
            pltpu.VMEM((8, ck), jnp.int32),
            pltpu.VMEM((8, ck), jnp.int32),
            pltpu.VMEM((4, ck, d), jnp.float32),
            pltpu.VMEM_SHARED((n, d), jnp.float32),
        ] + [pltpu.SemaphoreType.DMA] * 24,
    )
    def seg_kernel(table_hbm, src_hbm, dst_hbm, zeros_hbm, out_hbm,
                   idxg_v, idxs_v, rows_v, acc_sh, *sems):
        c = lax.axis_index("c")
        s = lax.axis_index("s")
        wid = s * _NC + c
        base = wid * per_w
        gsem = sems[0:4]
        ssem = sems[4:8]
        igsem = sems[8:16]
        issem = sems[16:24]

        def ldg(ci, sl):
            return pltpu.make_async_copy(
                src_hbm.at[pl.ds(base + ci * ck, ck)],
                idxg_v.at[sl], igsem[sl])

        def lds(ci, sl):
            return pltpu.make_async_copy(
                dst_hbm.at[pl.ds(base + ci * ck, ck)],
                idxs_v.at[sl], issem[sl])

        def gather(sl, b):
            return pltpu.make_async_copy(
                table_hbm.at[idxg_v.at[sl]], rows_v.at[b], gsem[b])

        def scatter(sl, b):
            return pltpu.make_async_copy(
                rows_v.at[b], acc_sh.at[idxs_v.at[sl]], ssem[b])

        @pl.when(s < nzt)
        def _():
            pltpu.sync_copy(zeros_hbm,
                            acc_sh.at[pl.ds(s * rows_per_tile, rows_per_tile)])
        for u in range(6):
            ldg(u, u).start()
            lds(u, u).start()
        for b in range(2):
            ldg(b, b).wait()
            lds(b, b).wait()
            gather(b, b).start()
        plsc.subcore_barrier()

        @pl.loop(0, (nchunks + 7) // 8)
        def _(cq):
            for u in range(8):
                ci = cq * 8 + u
                rb = u % 4
                nrb = (u + 2) % 4
                nsl = (u + 2) % 8
                psl = (u + 6) % 8

                @pl.when(ci < nchunks)
                def _():
                    gather(u, rb).wait()
                    scatter(u, rb).start(add=True)

                @pl.when(ci + 2 < nchunks)
                def _():
                    @pl.when(ci >= 2)
                    def _():
                        scatter(psl, nrb).wait()
                    ldg(ci + 2, nsl).wait()
                    lds(ci + 2, nsl).wait()
                    gather(nsl, nrb).start()

                    @pl.when(ci + 6 < nchunks)
                    def _():
                        ldg(ci + 6, psl).start()
                        lds(ci + 6, psl).start()

        for ci in range(nchunks - 4, nchunks):
            scatter(ci % 8, ci % 4).wait()
        plsc.subcore_barrier()

        @pl.when(s < nzt)
        def _():
            pltpu.sync_copy(acc_sh.at[pl.ds(s * rows_per_tile, rows_per_tile)],
                            out_hbm.at[c, pl.ds(s * rows_per_tile, rows_per_tile)])

    out = seg_kernel(table, src, dst, zeros)
    return out[0], out[1]


def _seg_sum_and_edge_gather(ts, t2, src, dst, n):
    d = ts.shape[1]
    e = src.shape[0]
    per_w = e // _NW
    nchunks = per_w // _CHUNK
    nzt = 10
    rows_per_tile = n // nzt
    zeros = jnp.zeros((rows_per_tile, d), jnp.float32)

    @functools.partial(
        pl.kernel,
        out_type=(jax.ShapeDtypeStruct((_NC, n, d), jnp.float32),
                  jax.ShapeDtypeStruct((e, d), jnp.float32)),
        mesh=_sc_mesh(),
        scratch_types=[
            pltpu.VMEM((4, _CHUNK), jnp.int32),
            pltpu.VMEM((4, _CHUNK), jnp.int32),
            pltpu.VMEM((2, _CHUNK, d), jnp.float32),
            pltpu.VMEM((2, _CHUNK, d), jnp.float32),
            pltpu.VMEM_SHARED((n, d), jnp.float32),
        ] + [pltpu.SemaphoreType.DMA] * 16,
    )
    def fused_kernel(ts_hbm, t2_hbm, src_hbm, dst_hbm, zeros_hbm,
                     out_hbm, t0x_hbm,
                     idxg_v, idxs_v, r1_v, r2_v, acc_sh, *sems):
        c = lax.axis_index("c")
        s = lax.axis_index("s")
        wid = s * _NC + c
        base = wid * per_w
        g1sem = sems[0:2]
        g2sem = sems[2:4]
        ssem = sems[4:6]
        wsem = sems[6:8]
        igsem = sems[8:12]
        issem = sems[12:16]

        def ldg(ci, sl):
            return pltpu.make_async_copy(
                src_hbm.at[pl.ds(base + ci * _CHUNK, _CHUNK)],
                idxg_v.at[sl], igsem[sl])

        def lds(ci, sl):
            return pltpu.make_async_copy(
                dst_hbm.at[pl.ds(base + ci * _CHUNK, _CHUNK)],
                idxs_v.at[sl], issem[sl])

        def gather1(sl, b):
            return pltpu.make_async_copy(
                ts_hbm.at[idxg_v.at[sl]], r1_v.at[b], g1sem[b])

        def gather2(sl, b):
            return pltpu.make_async_copy(
                t2_hbm.at[idxs_v.at[sl]], r2_v.at[b], g2sem[b])

        def scatter(sl, b):
            return pltpu.make_async_copy(
                r1_v.at[b], acc_sh.at[idxs_v.at[sl]], ssem[b])

        def twrite(ci, b):
            return pltpu.make_async_copy(
                r2_v.at[b], t0x_hbm.at[pl.ds(base + ci * _CHUNK, _CHUNK)],
                wsem[b])

        @pl.when(s < nzt)
        def _():
            pltpu.sync_copy(zeros_hbm,
                            acc_sh.at[pl.ds(s * rows_per_tile, rows_per_tile)])
        for u in range(4):
            ldg(u, u).start()
            lds(u, u).start()
        for b in range(2):
            ldg(b, b).wait()
            lds(b, b).wait()
            gather1(b, b).start()
            gather2(b, b).start()
        plsc.subcore_barrier()

        @pl.loop(0, (nchunks + 3) // 4)
        def _(cq):
            for u in range(4):
                ci = cq * 4 + u
                b = u % 2
                ob = 1 - b
                osl = (u + 1) % 4
                psl = (u + 3) % 4

                @pl.when(ci < nchunks)
                def _():
                    gather1(u, b).wait()
                    scatter(u, b).start(add=True)
                    gather2(u, b).wait()

                @pl.when(jnp.logical_and(ci >= 1, ci + 1 < nchunks))
                def _():
                    scatter(psl, ob).wait()
                    twrite(ci - 1, ob).wait()
                    ldg(ci + 1, osl).wait()
                    lds(ci + 1, osl).wait()
                    gather1(osl, ob).start()
                    gather2(osl, ob).start()

                    @pl.when(ci + 3 < nchunks)
                    def _():
                        ldg(ci + 3, psl).start()
                        lds(ci + 3, psl).start()

                @pl.when(ci < nchunks)
                def _():
                    @pl.loop(0, _CHUNK, unroll=8)
                    def _(ri):
                        for j in range(d // 2 // 16):
                            lo = pl.ds(j * 16, 16)
                            hi = pl.ds(d // 2 + j * 16, 16)
                            r2_v[b, ri, lo] = r1_v[b, ri, hi] + r2_v[b, ri, lo]

                    twrite(ci, b).start()

        for ci in (nchunks - 2, nchunks - 1):
            b = ci % 2
            scatter(ci % 4, b).wait()
            twrite(ci, b).wait()
        plsc.subcore_barrier()

        @pl.when(s < nzt)
        def _():
            pltpu.sync_copy(acc_sh.at[pl.ds(s * rows_per_tile, rows_per_tile)],
                            out_hbm.at[c, pl.ds(s * rows_per_tile, rows_per_tile)])

    return fused_kernel(ts, t2, src, dst, zeros)


def _gather_mul_seg_sum(kpad, p, src, dst, n):
    d = kpad.shape[1]
    dp = p.shape[1]
    e = src.shape[0]
    per_w = e // _NW
    nchunks = per_w // _DCHUNK
    nzt = 10
    rows_per_tile = n // nzt
    zeros = jnp.zeros((rows_per_tile, d), jnp.float32)

    @functools.partial(
        pl.kernel,
        out_type=jax.ShapeDtypeStruct((_NC, n, d), jnp.float32),
        mesh=_sc_mesh(),
        scratch_types=[
            pltpu.VMEM((8, _DCHUNK), jnp.int32),
            pltpu.VMEM((8, _DCHUNK), jnp.int32),
            pltpu.VMEM((4, _DCHUNK, d), jnp.float32),
            pltpu.VMEM((4, _DCHUNK, dp), jnp.float32),
            pltpu.VMEM_SHARED((n, d), jnp.float32),
        ] + [pltpu.SemaphoreType.DMA] * 28,
    )
    def gm_kernel(k_hbm, p_hbm, src_hbm, dst_hbm, zeros_hbm, out_hbm,
                  idxg_v, idxs_v, rows_v, pv_v, acc_sh, *sems):
        c = lax.axis_index("c")
        s = lax.axis_index("s")
        wid = s * _NC + c
        base = wid * per_w
        ksem = sems[0:4]
        psem = sems[4:8]
        ssem = sems[8:12]
        igsem = sems[12:20]
        issem = sems[20:28]

        def ldg(ci, sl):
            return pltpu.make_async_copy(
                src_hbm.at[pl.ds(base + ci * _DCHUNK, _DCHUNK)],
                idxg_v.at[sl], igsem[sl])

        def lds(ci, sl):
            return pltpu.make_async_copy(
                dst_hbm.at[pl.ds(base + ci * _DCHUNK, _DCHUNK)],
                idxs_v.at[sl], issem[sl])

        def gather(sl, b):
            return pltpu.make_async_copy(
                k_hbm.at[idxg_v.at[sl]], rows_v.at[b], ksem[b])

        def pread(ci, b):
            return pltpu.make_async_copy(
                p_hbm.at[pl.ds(base + ci * _DCHUNK, _DCHUNK)], pv_v.at[b],
                psem[b])

        def scatter(sl, b):
            return pltpu.make_async_copy(
                rows_v.at[b], acc_sh.at[idxs_v.at[sl]], ssem[b])

        @pl.when(s < nzt)
        def _():
            pltpu.sync_copy(zeros_hbm,
                            acc_sh.at[pl.ds(s * rows_per_tile, rows_per_tile)])
        for u in range(6):
            ldg(u, u).start()
            lds(u, u).start()
        for b in range(2):
            ldg(b, b).wait()
            lds(b, b).wait()
            gather(b, b).start()
            pread(b, b).start()
        plsc.subcore_barrier()

        @pl.loop(0, (nchunks + 7) // 8)
        def _(cq):
            for u in range(8):
                ci = cq * 8 + u
                rb = u % 4
                nrb = (u + 2) % 4
                nsl = (u + 2) % 8
                psl = (u + 6) % 8

                @pl.when(ci < nchunks)
                def _():
                    gather(u, rb).wait()
                    pread(ci, rb).wait()

                @pl.when(ci + 2 < nchunks)
                def _():
                    @pl.when(ci >= 2)
                    def _():
                        scatter(psl, nrb).wait()
                    ldg(ci + 2, nsl).wait()
                    lds(ci + 2, nsl).wait()
                    gather(nsl, nrb).start()
                    pread(ci + 2, nrb).start()

                    @pl.when(ci + 6 < nchunks)
                    def _():
                        ldg(ci + 6, psl).start()
                        lds(ci + 6, psl).start()

                @pl.when(ci < nchunks)
                def _():
                    @pl.loop(0, _DCHUNK, unroll=8)
                    def _(ri):
                        for j in range(dp // 16):
                            sl = pl.ds(j * 16, 16)
                            rows_v[rb, ri, sl] = (
                                rows_v[rb, ri, sl] * pv_v[rb, ri, sl])

                    scatter(u, rb).start(add=True)

        for ci in range(nchunks - 4, nchunks):
            scatter(ci % 8, ci % 4).wait()
        plsc.subcore_barrier()

        @pl.when(s < nzt)
        def _():
            pltpu.sync_copy(acc_sh.at[pl.ds(s * rows_per_tile, rows_per_tile)],
                            out_hbm.at[c, pl.ds(s * rows_per_tile, rows_per_tile)])

    out = gm_kernel(kpad, p, src, dst, zeros)
    return out[0], out[1]


def kernel(x, src, dst, encK_W, encK_b, encP1_W, encP1_b, encP2_W, encP2_b,
           Kf0_W, Kf0_b, Kf1_W, Kf1_b, Kf2_W, Kf2_b,
           Uf0_W, Uf0_b, Uf1_W, Uf1_b, Uf2_W, Uf2_b, H_W, H_b, D_W, D_b):
    n, in_dim = x.shape

    WB = encK_W @ Kf0_W
    bB = encK_b @ Kf0_W
    W1 = encP1_W @ Uf0_W
    b1 = encP1_b @ Uf0_W + Uf0_b
    W2 = encP2_W @ Uf0_W
    b2 = encP2_b @ Uf0_W
    Wcat = jnp.concatenate(
        [WB, W1, W2, jnp.zeros_like(W2)], axis=1)
    bcat = jnp.concatenate([bB, b1, b2, jnp.zeros_like(b2)], axis=0)

    pre_all = _tc_mm_bias(x, Wcat, bcat)
    ts = lax.slice(pre_all, (0, 0), (n, 128))
    t2 = lax.slice(pre_all, (0, 128), (n, 256))

    part, t0x = _seg_sum_and_edge_gather(ts, t2, src, dst, n)
    kpad = _tc_node_mlp(part[0], part[1],
                        Kf0_b, Kf1_W, Kf1_b, Kf2_W, Kf2_b)

    P = _tc_edge_mlp(t0x, Uf1_W, Uf1_b, Uf2_W, Uf2_b)

    e0, e1 = _gather_mul_seg_sum(kpad, P, src, dst, n)
    dd, pre = _tc_final_dense(e0, e1, H_W, H_b, D_W, D_b)

    d0, d1 = _seg_sum(dd, src, dst, n)
    return _tc_combine(pre, d0, d1)
